# Initial kernel scaffold; baseline (speedup 1.0000x reference)
import jax
import jax.numpy as jnp
from jax.experimental import pallas as pl


def kernel(h, idx):
    raise NotImplementedError("write your pallas kernel here")


# speedup vs baseline: 2.8711x; 2.8711x over previous
import functools

import jax
import jax.numpy as jnp
from jax import lax
from jax.experimental import pallas as pl
from jax.experimental.pallas import tpu as pltpu
from jax.experimental.pallas import tpu_sc as plsc

N_NODES = 100000
DIM = 128
N_EDGES = 500000
R = 2.0
T = 1.0
EPS = 1e-7

NC, NS, L = 2, 16, 16
NW = NC * NS
BC = 128
NCHUNK = 123
PER_W = BC * NCHUNK
P_PAD = PER_W * NW


def _sc_minkowski(h, idx0, idx1):
    mesh = plsc.VectorSubcoreMesh(core_axis_name="c", subcore_axis_name="s")

    @functools.partial(
        pl.kernel,
        out_type=jax.ShapeDtypeStruct((P_PAD,), jnp.float32),
        mesh=mesh,
        compiler_params=pltpu.CompilerParams(needs_layout_passes=False),
        scratch_types=[
            pltpu.VMEM((BC,), jnp.int32),
            pltpu.VMEM((BC,), jnp.int32),
            pltpu.VMEM((BC, DIM), jnp.float32),
            pltpu.VMEM((BC, DIM), jnp.float32),
            pltpu.VMEM((BC,), jnp.float32),
            pltpu.SemaphoreType.DMA,
            pltpu.SemaphoreType.DMA,
        ],
    )
    def k(h_hbm, i0_hbm, i1_hbm, out_hbm, i0_v, i1_v, ra_v, rb_v, out_v,
          sem_a, sem_b):
        wid = lax.axis_index("s") * NC + lax.axis_index("c")
        base = wid * PER_W
        lane = lax.iota(jnp.int32, L)
        w0 = jnp.where(lane == 0, -1.0, 1.0).astype(jnp.float32)

        def chunk_body(kk, _):
            off = base + kk * BC
            pltpu.sync_copy(i0_hbm.at[pl.ds(off, BC)], i0_v)
            pltpu.sync_copy(i1_hbm.at[pl.ds(off, BC)], i1_v)
            ca = pltpu.async_copy(h_hbm.at[i0_v], ra_v, sem_a)
            cb = pltpu.async_copy(h_hbm.at[i1_v], rb_v, sem_b)
            ca.wait()
            cb.wait()

            for g in range(BC // L):
                def pair_body(j, res, g=g):
                    i = g * L + j
                    acc = ra_v[i, pl.ds(0, L)] * rb_v[i, pl.ds(0, L)] * w0
                    for c in range(1, DIM // L):
                        acc = acc + ra_v[i, pl.ds(c * L, L)] * rb_v[i, pl.ds(c * L, L)]
                    return jnp.where(lane == j, jnp.sum(acc), res)

                out_v[pl.ds(g * L, L)] = lax.fori_loop(
                    0, L, pair_body, jnp.zeros((L,), jnp.float32))
            pltpu.sync_copy(out_v, out_hbm.at[pl.ds(off, BC)])
            return 0

        lax.fori_loop(0, NCHUNK, chunk_body, 0)

    return k(h, idx0, idx1)


def _decode_body(p_ref, o_ref):
    prod = p_ref[...]
    theta = jnp.maximum(-prod, 1.0 + EPS)
    dist = jnp.log(theta + jnp.sqrt(theta - 1.0) * jnp.sqrt(theta + 1.0))
    sqdist = jnp.minimum(dist * dist, 50.0)
    o_ref[...] = 1.0 / (jnp.exp((sqdist - R) / T) + 1.0)


def _tc_decode(prod):
    rows = P_PAD // DIM
    return pl.pallas_call(
        _decode_body,
        out_shape=jax.ShapeDtypeStruct((rows, DIM), jnp.float32),
    )(prod.reshape(rows, DIM)).reshape(-1)


def kernel(h, idx):
    pad = P_PAD - N_EDGES
    idx0 = jnp.concatenate([idx[:, 0], jnp.zeros((pad,), jnp.int32)])
    idx1 = jnp.concatenate([idx[:, 1], jnp.zeros((pad,), jnp.int32)])
    prod = _sc_minkowski(h, idx0, idx1)
    return _tc_decode(prod)[:N_EDGES]
